# baseline (speedup 1.0000x reference)
import functools

import jax
import jax.numpy as jnp
from jax import lax
from jax.experimental import pallas as pl
from jax.experimental.pallas import tpu as pltpu
from jax.experimental.pallas import tpu_sc as plsc

N = 10000
D = 128
E = 320000
EL = 16384
NC, NS = 2, 16
CB = 128
SUP = 4
CPT = 160
NSUP = CPT // SUP
E_PAD = NS * CPT * CB
N_PAD = 10112
RPT = N_PAD // NS
RPT_F = (RPT // CB) * CB
RPT_T = RPT - RPT_F

_MESH = plsc.VectorSubcoreMesh(core_axis_name="c", subcore_axis_name="s")


def _agg_body(tab_a, tab_b, src_hbm, dst_hbm, zrow_hbm,
              agg_out,
              src_slab, dst_slab, rows0, rows1, rows2, acc,
              sem0, sem1, ssem0, ssem1):
    cid = lax.axis_index("c")
    sid = lax.axis_index("s")
    base = sid * RPT
    bufs = (rows0, rows1, rows2)
    sems = (sem0, sem1)

    pltpu.sync_copy(zrow_hbm, rows0)
    for i in range(RPT // CB):
        pltpu.sync_copy(rows0, acc.at[pl.ds(base + i * CB, CB)])
    pltpu.sync_copy(rows0.at[pl.ds(0, RPT_T)],
                    acc.at[pl.ds(base + RPT_F, RPT_T)])
    plsc.subcore_barrier()

    ssems = (ssem0, ssem1)

    def run(tab, gslab, sslab):
        def sup(s2, carry):
            pltpu.sync_copy(src_hbm.at[sid, pl.ds(s2 * SUP, SUP)], src_slab)
            pltpu.sync_copy(dst_hbm.at[sid, pl.ds(s2 * SUP, SUP)], dst_slab)
            cps = [pltpu.async_copy(tab.at[gslab.at[0]], bufs[0], sems[0]),
                   pltpu.async_copy(tab.at[gslab.at[1]], bufs[1], sems[1])]
            scps = []
            for k in range(SUP):
                cps[k % 2].wait()
                if k + 2 < SUP:
                    if k >= 1:
                        scps[k - 1].wait()
                    cps[k % 2] = pltpu.async_copy(
                        tab.at[gslab.at[k + 2]], bufs[(k + 2) % 3],
                        sems[k % 2])
                scps.append(pltpu.async_copy(
                    bufs[k % 3], acc.at[sslab.at[k]], ssems[k % 2],
                    add=True))
            for k in range(max(0, SUP - 3), SUP):
                scps[k].wait()
            return carry
        lax.fori_loop(0, NSUP, sup, 0)

    @pl.when(cid == 0)
    def _():
        run(tab_a, src_slab, dst_slab)

    @pl.when(cid == 1)
    def _():
        run(tab_b, dst_slab, src_slab)

    plsc.subcore_barrier()
    for i in range(RPT // CB):
        pltpu.sync_copy(acc.at[pl.ds(base + i * CB, CB)], rows0)
        pltpu.sync_copy(rows0, agg_out.at[cid, pl.ds(base + i * CB, CB)])
    pltpu.sync_copy(acc.at[pl.ds(base + RPT_F, RPT_T)],
                    rows0.at[pl.ds(0, RPT_T)])
    pltpu.sync_copy(rows0.at[pl.ds(0, RPT_T)],
                    agg_out.at[cid, pl.ds(base + RPT_F, RPT_T)])


_sc_agg = pl.kernel(
    _agg_body,
    out_type=jax.ShapeDtypeStruct((NC, N_PAD, D), jnp.float32),
    mesh=_MESH,
    scratch_types=(
        pltpu.VMEM((SUP, CB), jnp.int32),
        pltpu.VMEM((SUP, CB), jnp.int32),
        pltpu.VMEM((CB, D), jnp.float32),
        pltpu.VMEM((CB, D), jnp.float32),
        pltpu.VMEM((CB, D), jnp.float32),
        pltpu.VMEM_SHARED((N_PAD, D), jnp.float32),
        pltpu.SemaphoreType.DMA,
        pltpu.SemaphoreType.DMA,
        pltpu.SemaphoreType.DMA,
        pltpu.SemaphoreType.DMA,
    ),
)


def _cnt_body(islab_hbm, zrow_hbm, ones_hbm, cnt_out, slab, rows, acc):
    cid = lax.axis_index("c")
    sid = lax.axis_index("s")
    base = sid * RPT

    pltpu.sync_copy(zrow_hbm, rows)
    for i in range(RPT // CB):
        pltpu.sync_copy(rows, acc.at[pl.ds(base + i * CB, CB)])
    pltpu.sync_copy(rows.at[pl.ds(0, RPT_T)],
                    acc.at[pl.ds(base + RPT_F, RPT_T)])
    pltpu.sync_copy(ones_hbm, rows)
    plsc.subcore_barrier()

    def sup(s2, carry):
        pltpu.sync_copy(islab_hbm.at[cid, sid, pl.ds(s2 * SUP, SUP)], slab)
        for k in range(SUP):
            pltpu.sync_copy(rows, acc.at[slab.at[k]], add=True)
        return carry
    lax.fori_loop(0, NSUP, sup, 0)

    plsc.subcore_barrier()
    for i in range(RPT // CB):
        pltpu.sync_copy(acc.at[pl.ds(base + i * CB, CB)], rows)
        pltpu.sync_copy(rows, cnt_out.at[cid, pl.ds(base + i * CB, CB)])
    pltpu.sync_copy(acc.at[pl.ds(base + RPT_F, RPT_T)],
                    rows.at[pl.ds(0, RPT_T)])
    pltpu.sync_copy(rows.at[pl.ds(0, RPT_T)],
                    cnt_out.at[cid, pl.ds(base + RPT_F, RPT_T)])


_sc_cnt = pl.kernel(
    _cnt_body,
    out_type=jax.ShapeDtypeStruct((NC, N_PAD, D), jnp.float32),
    mesh=_MESH,
    scratch_types=(
        pltpu.VMEM((SUP, CB), jnp.int32),
        pltpu.VMEM((CB, D), jnp.float32),
        pltpu.VMEM_SHARED((N_PAD, D), jnp.float32),
    ),
)


def _cls_body(ou_hbm, op_hbm, l0_hbm, l1_hbm, zrow_hbm, pred_out,
              l0s, l1s, u_rows, p_rows, res, sem):
    cid = lax.axis_index("c")
    sid = lax.axis_index("s")
    wid = cid * NS + sid
    per_tile = EL // (NC * NS)
    n_chunks = per_tile // CB

    pltpu.sync_copy(l0_hbm.at[wid], l0s)
    pltpu.sync_copy(l1_hbm.at[wid], l1s)
    pltpu.sync_copy(zrow_hbm, res)

    def chunk(jj, carry):
        pltpu.async_copy(ou_hbm.at[l0s.at[jj]], u_rows, sem).wait()
        pltpu.async_copy(op_hbm.at[l1s.at[jj]], p_rows, sem).wait()

        def row(r, c2):
            acc16 = jnp.zeros((16,), jnp.float32)
            for c in range(D // 16):
                acc16 = acc16 + (u_rows[r, pl.ds(c * 16, 16)] *
                                 p_rows[r, pl.ds(c * 16, 16)])
            res[r, pl.ds(0, 16)] = acc16
            return c2
        lax.fori_loop(0, CB, row, 0)
        pltpu.sync_copy(
            res, pred_out.at[pl.ds(wid * per_tile + jj * CB, CB)])
        return carry
    lax.fori_loop(0, n_chunks, chunk, 0)


_sc_classifier = pl.kernel(
    _cls_body,
    out_type=jax.ShapeDtypeStruct((EL, D), jnp.float32),
    mesh=_MESH,
    scratch_types=(
        pltpu.VMEM((EL // (NC * NS) // CB, CB), jnp.int32),
        pltpu.VMEM((EL // (NC * NS) // CB, CB), jnp.int32),
        pltpu.VMEM((CB, D), jnp.float32),
        pltpu.VMEM((CB, D), jnp.float32),
        pltpu.VMEM((CB, D), jnp.float32),
        pltpu.SemaphoreType.DMA,
    ),
)

_RB = 632
_row_spec = pl.BlockSpec((_RB, D), lambda i: (i, 0))
_w_spec = pl.BlockSpec((D, D), lambda i: (0, 0))
_b_spec = pl.BlockSpec((1, D), lambda i: (0, 0))
_f_spec = pl.BlockSpec((1, 1), lambda i: (0, 0))


def _fin_body(x, o):
    o[...] = jnp.sum(x[...][:, :16], axis=1, keepdims=True)


def _tc_finish(pred16):
    out = pl.pallas_call(
        _fin_body,
        grid=(EL // 2048,),
        in_specs=[pl.BlockSpec((2048, D), lambda i: (i, 0))],
        out_specs=pl.BlockSpec((2048, 1), lambda i: (i, 0)),
        out_shape=jax.ShapeDtypeStruct((EL, 1), jnp.float32),
    )(pred16)
    return out.reshape(EL)


def _pre_body(xu, xp, wa, wb, la, lb):
    la[...] = jnp.dot(xu[...], wa[...], preferred_element_type=jnp.float32)
    lb[...] = jnp.dot(xp[...], wb[...], preferred_element_type=jnp.float32)


def _tc_pre(xu, xp, wa, wb):
    return pl.pallas_call(
        _pre_body,
        grid=(N_PAD // _RB,),
        in_specs=[_row_spec, _row_spec, _w_spec, _w_spec],
        out_specs=[_row_spec, _row_spec],
        out_shape=[jax.ShapeDtypeStruct((N_PAD, D), jnp.float32)] * 2,
    )(xu, xp, wa, wb)


def _stage_body(ap, cp, au, cu, rp, ru, wrp, wru, wna, wnb, bp, bu, flag,
                la2, lb2, hp, hu):
    s = flag[0, 0]
    mp = ap[...] / jnp.maximum(cp[...], 1.0)
    mu = au[...] / jnp.maximum(cu[...], 1.0)
    zp = mp + bp[...] + jnp.dot(rp[...], wrp[...],
                                preferred_element_type=jnp.float32)
    zu = mu + bu[...] + jnp.dot(ru[...], wru[...],
                                preferred_element_type=jnp.float32)
    hp_v = jnp.maximum(zp, s * zp)
    hu_v = jnp.maximum(zu, s * zu)
    hp[...] = hp_v
    hu[...] = hu_v
    la2[...] = jnp.dot(hu_v, wna[...], preferred_element_type=jnp.float32)
    lb2[...] = jnp.dot(hp_v, wnb[...], preferred_element_type=jnp.float32)


def _tc_stage(ap, cp, au, cu, rp, ru, wrp, wru, wna, wnb, bp, bu, flag):
    return pl.pallas_call(
        _stage_body,
        grid=(N_PAD // _RB,),
        in_specs=[_row_spec, _row_spec, _row_spec, _row_spec,
                  _row_spec, _row_spec,
                  _w_spec, _w_spec, _w_spec, _w_spec,
                  _b_spec, _b_spec, _f_spec],
        out_specs=[_row_spec] * 4,
        out_shape=[jax.ShapeDtypeStruct((N_PAD, D), jnp.float32)] * 4,
    )(ap, cp, au, cu, rp, ru, wrp, wru, wna, wnb, bp, bu, flag)


def kernel(x_user, x_product, edge_index, edge_label_index,
           W1_buys_l, W1_buys_r, W1_rev_l, W1_rev_r,
           W2_buys_l, W2_buys_r, W2_rev_l, W2_rev_r,
           b1_buys, b1_rev, b2_buys, b2_rev):
    f32 = jnp.float32
    xu = jnp.zeros((N_PAD, D), f32).at[:N].set(x_user.astype(f32))
    xp = jnp.zeros((N_PAD, D), f32).at[:N].set(x_product.astype(f32))

    ei = edge_index.astype(jnp.int32)
    pad = jnp.full((E_PAD - E,), N, jnp.int32)
    src = jnp.concatenate([ei[0], pad]).reshape(NS, CPT, CB)
    dst = jnp.concatenate([ei[1], pad]).reshape(NS, CPT, CB)

    zrow = jnp.zeros((CB, D), f32)
    ones_rows = jnp.ones((CB, D), f32)
    islab = jnp.stack([dst, src])

    cntw = _sc_cnt(islab, zrow, ones_rows)
    la0, lb0 = _tc_pre(xu, xp, W1_buys_l, W1_rev_l)

    wrp_s = jnp.stack([W1_buys_r, W2_buys_r])
    wru_s = jnp.stack([W1_rev_r, W2_rev_r])
    wzero = jnp.zeros((D, D), f32)
    wna_s = jnp.stack([W2_buys_l, wzero])
    wnb_s = jnp.stack([W2_rev_l, wzero])
    bp_s = jnp.stack([b1_buys.reshape(1, D), b2_buys.reshape(1, D)])
    bu_s = jnp.stack([b1_rev.reshape(1, D), b2_rev.reshape(1, D)])
    flag_s = jnp.array([0.0, 1.0], f32).reshape(2, 1, 1)

    def body(carry, xs):
        la, lb, rp, ru = carry
        wrp, wru, wna, wnb, bp, bu, flag = xs
        agg = _sc_agg(la, lb, src, dst, zrow)
        la2, lb2, hp, hu = _tc_stage(agg[0], cntw[0], agg[1], cntw[1], rp, ru,
                                     wrp, wru, wna, wnb, bp, bu, flag)
        return (la2, lb2, hp, hu), None

    (_, _, o_prod, o_user), _ = lax.scan(
        body, (la0, lb0, xp, xu),
        (wrp_s, wru_s, wna_s, wnb_s, bp_s, bu_s, flag_s))

    eli = edge_label_index.astype(jnp.int32)
    l0 = eli[0].reshape(NC * NS, EL // (NC * NS) // CB, CB)
    l1 = eli[1].reshape(NC * NS, EL // (NC * NS) // CB, CB)
    pred16 = _sc_classifier(o_user, o_prod, l0, l1, zrow)
    return _tc_finish(pred16)


# speedup vs baseline: 1.0043x; 1.0043x over previous
import functools

import jax
import jax.numpy as jnp
from jax import lax
from jax.experimental import pallas as pl
from jax.experimental.pallas import tpu as pltpu
from jax.experimental.pallas import tpu_sc as plsc

N = 10000
D = 128
E = 320000
EL = 16384
NC, NS = 2, 16
CB = 128
SUP = 4
CPT = 160
NSUP = CPT // SUP
E_PAD = NS * CPT * CB
N_PAD = 10112
RPT = N_PAD // NS
RPT_F = (RPT // CB) * CB
RPT_T = RPT - RPT_F

_MESH = plsc.VectorSubcoreMesh(core_axis_name="c", subcore_axis_name="s")


def _agg_body(tab_a, tab_b, src_hbm, dst_hbm, zrow_hbm,
              agg_out,
              src_slab, dst_slab, rows0, rows1, rows2, acc,
              sem0, sem1, ssem0, ssem1):
    cid = lax.axis_index("c")
    sid = lax.axis_index("s")
    base = sid * RPT
    bufs = (rows0, rows1, rows2)
    sems = (sem0, sem1)

    pltpu.sync_copy(zrow_hbm, rows0)
    for i in range(RPT // CB):
        pltpu.sync_copy(rows0, acc.at[pl.ds(base + i * CB, CB)])
    pltpu.sync_copy(rows0.at[pl.ds(0, RPT_T)],
                    acc.at[pl.ds(base + RPT_F, RPT_T)])
    plsc.subcore_barrier()

    ssems = (ssem0, ssem1)

    def run(tab, gslab, sslab):
        def sup(s2, carry):
            pltpu.sync_copy(src_hbm.at[sid, pl.ds(s2 * SUP, SUP)], src_slab)
            pltpu.sync_copy(dst_hbm.at[sid, pl.ds(s2 * SUP, SUP)], dst_slab)
            cps = [pltpu.async_copy(tab.at[gslab.at[0]], bufs[0], sems[0]),
                   pltpu.async_copy(tab.at[gslab.at[1]], bufs[1], sems[1])]
            scps = []
            for k in range(SUP):
                cps[k % 2].wait()
                if k + 2 < SUP:
                    if k >= 1:
                        scps[k - 1].wait()
                    cps[k % 2] = pltpu.async_copy(
                        tab.at[gslab.at[k + 2]], bufs[(k + 2) % 3],
                        sems[k % 2])
                scps.append(pltpu.async_copy(
                    bufs[k % 3], acc.at[sslab.at[k]], ssems[k % 2],
                    add=True))
            for k in range(max(0, SUP - 3), SUP):
                scps[k].wait()
            return carry
        lax.fori_loop(0, NSUP, sup, 0)

    @pl.when(cid == 0)
    def _():
        run(tab_a, src_slab, dst_slab)

    @pl.when(cid == 1)
    def _():
        run(tab_b, dst_slab, src_slab)

    plsc.subcore_barrier()
    for i in range(RPT // CB):
        pltpu.sync_copy(acc.at[pl.ds(base + i * CB, CB)], rows0)
        pltpu.sync_copy(rows0, agg_out.at[cid, pl.ds(base + i * CB, CB)])
    pltpu.sync_copy(acc.at[pl.ds(base + RPT_F, RPT_T)],
                    rows0.at[pl.ds(0, RPT_T)])
    pltpu.sync_copy(rows0.at[pl.ds(0, RPT_T)],
                    agg_out.at[cid, pl.ds(base + RPT_F, RPT_T)])


_sc_agg = pl.kernel(
    _agg_body,
    out_type=jax.ShapeDtypeStruct((NC, N_PAD, D), jnp.float32),
    mesh=_MESH,
    scratch_types=(
        pltpu.VMEM((SUP, CB), jnp.int32),
        pltpu.VMEM((SUP, CB), jnp.int32),
        pltpu.VMEM((CB, D), jnp.float32),
        pltpu.VMEM((CB, D), jnp.float32),
        pltpu.VMEM((CB, D), jnp.float32),
        pltpu.VMEM_SHARED((N_PAD, D), jnp.float32),
        pltpu.SemaphoreType.DMA,
        pltpu.SemaphoreType.DMA,
        pltpu.SemaphoreType.DMA,
        pltpu.SemaphoreType.DMA,
    ),
)


def _cnt_body(islab_hbm, zrow_hbm, ones_hbm, cnt_out, slab, rows, acc,
              csem0, csem1):
    cid = lax.axis_index("c")
    sid = lax.axis_index("s")
    base = sid * RPT

    pltpu.sync_copy(zrow_hbm, rows)
    for i in range(RPT // CB):
        pltpu.sync_copy(rows, acc.at[pl.ds(base + i * CB, CB)])
    pltpu.sync_copy(rows.at[pl.ds(0, RPT_T)],
                    acc.at[pl.ds(base + RPT_F, RPT_T)])
    pltpu.sync_copy(ones_hbm, rows)
    plsc.subcore_barrier()

    csems = (csem0, csem1)

    def sup(s2, carry):
        pltpu.sync_copy(islab_hbm.at[cid, sid, pl.ds(s2 * SUP, SUP)], slab)
        cps = [pltpu.async_copy(rows, acc.at[slab.at[k]], csems[k % 2],
                                add=True)
               for k in range(SUP)]
        for cp in cps:
            cp.wait()
        return carry
    lax.fori_loop(0, NSUP, sup, 0)

    plsc.subcore_barrier()
    for i in range(RPT // CB):
        pltpu.sync_copy(acc.at[pl.ds(base + i * CB, CB)], rows)
        pltpu.sync_copy(rows, cnt_out.at[cid, pl.ds(base + i * CB, CB)])
    pltpu.sync_copy(acc.at[pl.ds(base + RPT_F, RPT_T)],
                    rows.at[pl.ds(0, RPT_T)])
    pltpu.sync_copy(rows.at[pl.ds(0, RPT_T)],
                    cnt_out.at[cid, pl.ds(base + RPT_F, RPT_T)])


_sc_cnt = pl.kernel(
    _cnt_body,
    out_type=jax.ShapeDtypeStruct((NC, N_PAD, D), jnp.float32),
    mesh=_MESH,
    scratch_types=(
        pltpu.VMEM((SUP, CB), jnp.int32),
        pltpu.VMEM((CB, D), jnp.float32),
        pltpu.VMEM_SHARED((N_PAD, D), jnp.float32),
        pltpu.SemaphoreType.DMA,
        pltpu.SemaphoreType.DMA,
    ),
)


def _cls_body(ou_hbm, op_hbm, l0_hbm, l1_hbm, zrow_hbm, pred_out,
              l0s, l1s, u_rows, p_rows, res, sem):
    cid = lax.axis_index("c")
    sid = lax.axis_index("s")
    wid = cid * NS + sid
    per_tile = EL // (NC * NS)
    n_chunks = per_tile // CB

    pltpu.sync_copy(l0_hbm.at[wid], l0s)
    pltpu.sync_copy(l1_hbm.at[wid], l1s)
    pltpu.sync_copy(zrow_hbm, res)

    def chunk(jj, carry):
        pltpu.async_copy(ou_hbm.at[l0s.at[jj]], u_rows, sem).wait()
        pltpu.async_copy(op_hbm.at[l1s.at[jj]], p_rows, sem).wait()

        def row(r, c2):
            acc16 = jnp.zeros((16,), jnp.float32)
            for c in range(D // 16):
                acc16 = acc16 + (u_rows[r, pl.ds(c * 16, 16)] *
                                 p_rows[r, pl.ds(c * 16, 16)])
            res[r, pl.ds(0, 16)] = acc16
            return c2
        lax.fori_loop(0, CB, row, 0)
        pltpu.sync_copy(
            res, pred_out.at[pl.ds(wid * per_tile + jj * CB, CB)])
        return carry
    lax.fori_loop(0, n_chunks, chunk, 0)


_sc_classifier = pl.kernel(
    _cls_body,
    out_type=jax.ShapeDtypeStruct((EL, D), jnp.float32),
    mesh=_MESH,
    scratch_types=(
        pltpu.VMEM((EL // (NC * NS) // CB, CB), jnp.int32),
        pltpu.VMEM((EL // (NC * NS) // CB, CB), jnp.int32),
        pltpu.VMEM((CB, D), jnp.float32),
        pltpu.VMEM((CB, D), jnp.float32),
        pltpu.VMEM((CB, D), jnp.float32),
        pltpu.SemaphoreType.DMA,
    ),
)

_RB = 632
_row_spec = pl.BlockSpec((_RB, D), lambda i: (i, 0))
_w_spec = pl.BlockSpec((D, D), lambda i: (0, 0))
_b_spec = pl.BlockSpec((1, D), lambda i: (0, 0))
_f_spec = pl.BlockSpec((1, 1), lambda i: (0, 0))


def _fin_body(x, o):
    o[...] = jnp.sum(x[...][:, :16], axis=1, keepdims=True)


def _tc_finish(pred16):
    out = pl.pallas_call(
        _fin_body,
        grid=(EL // 2048,),
        in_specs=[pl.BlockSpec((2048, D), lambda i: (i, 0))],
        out_specs=pl.BlockSpec((2048, 1), lambda i: (i, 0)),
        out_shape=jax.ShapeDtypeStruct((EL, 1), jnp.float32),
    )(pred16)
    return out.reshape(EL)


def _pre_body(xu, xp, wa, wb, la, lb):
    la[...] = jnp.dot(xu[...], wa[...], preferred_element_type=jnp.float32)
    lb[...] = jnp.dot(xp[...], wb[...], preferred_element_type=jnp.float32)


def _tc_pre(xu, xp, wa, wb):
    return pl.pallas_call(
        _pre_body,
        grid=(N_PAD // _RB,),
        in_specs=[_row_spec, _row_spec, _w_spec, _w_spec],
        out_specs=[_row_spec, _row_spec],
        out_shape=[jax.ShapeDtypeStruct((N_PAD, D), jnp.float32)] * 2,
    )(xu, xp, wa, wb)


def _stage_body(ap, cp, au, cu, rp, ru, wrp, wru, wna, wnb, bp, bu, flag,
                la2, lb2, hp, hu):
    s = flag[0, 0]
    mp = ap[...] / jnp.maximum(cp[...], 1.0)
    mu = au[...] / jnp.maximum(cu[...], 1.0)
    zp = mp + bp[...] + jnp.dot(rp[...], wrp[...],
                                preferred_element_type=jnp.float32)
    zu = mu + bu[...] + jnp.dot(ru[...], wru[...],
                                preferred_element_type=jnp.float32)
    hp_v = jnp.maximum(zp, s * zp)
    hu_v = jnp.maximum(zu, s * zu)
    hp[...] = hp_v
    hu[...] = hu_v
    la2[...] = jnp.dot(hu_v, wna[...], preferred_element_type=jnp.float32)
    lb2[...] = jnp.dot(hp_v, wnb[...], preferred_element_type=jnp.float32)


def _tc_stage(ap, cp, au, cu, rp, ru, wrp, wru, wna, wnb, bp, bu, flag):
    return pl.pallas_call(
        _stage_body,
        grid=(N_PAD // _RB,),
        in_specs=[_row_spec, _row_spec, _row_spec, _row_spec,
                  _row_spec, _row_spec,
                  _w_spec, _w_spec, _w_spec, _w_spec,
                  _b_spec, _b_spec, _f_spec],
        out_specs=[_row_spec] * 4,
        out_shape=[jax.ShapeDtypeStruct((N_PAD, D), jnp.float32)] * 4,
    )(ap, cp, au, cu, rp, ru, wrp, wru, wna, wnb, bp, bu, flag)


def kernel(x_user, x_product, edge_index, edge_label_index,
           W1_buys_l, W1_buys_r, W1_rev_l, W1_rev_r,
           W2_buys_l, W2_buys_r, W2_rev_l, W2_rev_r,
           b1_buys, b1_rev, b2_buys, b2_rev):
    f32 = jnp.float32
    xu = jnp.zeros((N_PAD, D), f32).at[:N].set(x_user.astype(f32))
    xp = jnp.zeros((N_PAD, D), f32).at[:N].set(x_product.astype(f32))

    ei = edge_index.astype(jnp.int32)
    pad = jnp.full((E_PAD - E,), N, jnp.int32)
    src = jnp.concatenate([ei[0], pad]).reshape(NS, CPT, CB)
    dst = jnp.concatenate([ei[1], pad]).reshape(NS, CPT, CB)

    zrow = jnp.zeros((CB, D), f32)
    ones_rows = jnp.ones((CB, D), f32)
    islab = jnp.stack([dst, src])

    cntw = _sc_cnt(islab, zrow, ones_rows)
    la0, lb0 = _tc_pre(xu, xp, W1_buys_l, W1_rev_l)

    wrp_s = jnp.stack([W1_buys_r, W2_buys_r])
    wru_s = jnp.stack([W1_rev_r, W2_rev_r])
    wzero = jnp.zeros((D, D), f32)
    wna_s = jnp.stack([W2_buys_l, wzero])
    wnb_s = jnp.stack([W2_rev_l, wzero])
    bp_s = jnp.stack([b1_buys.reshape(1, D), b2_buys.reshape(1, D)])
    bu_s = jnp.stack([b1_rev.reshape(1, D), b2_rev.reshape(1, D)])
    flag_s = jnp.array([0.0, 1.0], f32).reshape(2, 1, 1)

    def body(carry, xs):
        la, lb, rp, ru = carry
        wrp, wru, wna, wnb, bp, bu, flag = xs
        agg = _sc_agg(la, lb, src, dst, zrow)
        la2, lb2, hp, hu = _tc_stage(agg[0], cntw[0], agg[1], cntw[1], rp, ru,
                                     wrp, wru, wna, wnb, bp, bu, flag)
        return (la2, lb2, hp, hu), None

    (_, _, o_prod, o_user), _ = lax.scan(
        body, (la0, lb0, xp, xu),
        (wrp_s, wru_s, wna_s, wnb_s, bp_s, bu_s, flag_s))

    eli = edge_label_index.astype(jnp.int32)
    l0 = eli[0].reshape(NC * NS, EL // (NC * NS) // CB, CB)
    l1 = eli[1].reshape(NC * NS, EL // (NC * NS) // CB, CB)
    pred16 = _sc_classifier(o_user, o_prod, l0, l1, zrow)
    return _tc_finish(pred16)
